# baseline (speedup 1.0000x reference)
import jax
import jax.numpy as jnp
from jax.experimental import pallas as pl
from jax.experimental.pallas import tpu as pltpu

N3 = 300000
N = N3 // 3
D = 128
B = 5000
NZ = N // B
F8 = jnp.float8_e4m3fn


def _body(z_ref, zp_ref, zn_ref, w_ref, o_ref,
          s_ref, sc_ref, cz_ref, cp_ref, cn_ref, q_ref, ls_ref):
    g = pl.program_id(0)

    @pl.when(g == 0)
    def _init():
        s_ref[...] = jnp.zeros_like(s_ref)

    ones_row = jnp.ones((1, D), jnp.bfloat16)
    negones_row = jnp.full((1, D), -1.0, jnp.bfloat16)

    def lanered(v, e):
        return jax.lax.dot_general(
            v, e, (((1,), (1,)), ((), ())),
            preferred_element_type=jnp.float32)

    def colsum(w, x):
        return jax.lax.dot_general(
            w, x, (((1,), (0,)), ((), ())),
            preferred_element_type=jnp.float32)

    @pl.when(g < NZ)
    def _phase_a():
        zb = z_ref[...].astype(jnp.bfloat16)
        zpb = zp_ref[...].astype(jnp.bfloat16)
        znb = zn_ref[...].astype(jnp.bfloat16)
        z8 = zb.astype(F8)
        zp8 = zpb.astype(F8)
        zn8 = znb.astype(F8)
        ls_ref[pl.ds(g, 1), :] = lanered(ones_row, zb * zpb)
        ls_ref[pl.ds(NZ + g, 1), :] = lanered(negones_row, zb * znb)
        s_ref[...] += (
            colsum(w_ref[pl.ds(g, 1), :].astype(F8), z8)
            + colsum(w_ref[pl.ds(NZ + g, 1), :].astype(F8), zp8)
            + colsum(w_ref[pl.ds(2 * NZ + g, 1), :].astype(F8), zn8))
        q_ref[pl.ds(g, 1), :] = lanered(ones_row, zb * zb)
        q_ref[pl.ds(NZ + g, 1), :] = lanered(ones_row, zpb * zpb)
        q_ref[pl.ds(2 * NZ + g, 1), :] = lanered(ones_row, znb * znb)
        cz_ref[g] = z8
        cp_ref[g] = zp8
        cn_ref[g] = zn8

    @pl.when(g == NZ)
    def _finish():
        v = ls_ref[...]
        lssum = jnp.sum(jnp.minimum(v, 0.0)
                        - jnp.log1p(jnp.exp(-jnp.abs(v))))

        wall = w_ref[...]
        cnt = jnp.maximum(jnp.sum(wall), 1.0)
        mu = s_ref[...] / cnt
        m = jnp.sum(mu * mu)
        mu8 = (mu * -128.0).astype(F8)

        for t, c_ref in enumerate((cz_ref, cp_ref, cn_ref)):
            for j in range(NZ):
                i = t * NZ + j
                q_ref[pl.ds(i, 1), :] += (
                    lanered(mu8, c_ref[j]) * (1.0 / 64.0))

        d2 = q_ref[...] + m
        coag = jnp.sum(jnp.sqrt(jnp.maximum(wall * d2, 0.0)))

        sig = 1.0 / (1.0 + jnp.exp(-coag))
        total = -lssum / N + sig - 0.5
        o_ref[...] = jnp.full((1, 1), total, dtype=jnp.float32)


def kernel(out, mask):
    w = mask.astype(jnp.float32).reshape(3 * NZ, B)

    def omap(t):
        return lambda g: (t * NZ + jnp.minimum(g, NZ - 1), 0)

    res = pl.pallas_call(
        _body,
        grid=(NZ + 1,),
        in_specs=[
            pl.BlockSpec((B, D), omap(0)),
            pl.BlockSpec((B, D), omap(1)),
            pl.BlockSpec((B, D), omap(2)),
            pl.BlockSpec((3 * NZ, B), lambda g: (0, 0)),
        ],
        out_specs=pl.BlockSpec((1, 1), lambda g: (0, 0)),
        out_shape=jax.ShapeDtypeStruct((1, 1), jnp.float32),
        scratch_shapes=[
            pltpu.VMEM((1, D), jnp.float32),
            pltpu.SMEM((4,), jnp.float32),
            pltpu.VMEM((NZ, B, D), F8),
            pltpu.VMEM((NZ, B, D), F8),
            pltpu.VMEM((NZ, B, D), F8),
            pltpu.VMEM((3 * NZ, B), jnp.float32),
            pltpu.VMEM((2 * NZ, B), jnp.float32),
        ],
        compiler_params=pltpu.CompilerParams(
            dimension_semantics=("arbitrary",),
        ),
    )(out, out, out, w)
    return res[0, 0]
